# baseline (speedup 1.0000x reference)
import functools

import jax
import jax.numpy as jnp
from jax import lax
from jax.experimental import pallas as pl
from jax.experimental.pallas import tpu as pltpu
from jax.experimental.pallas import tpu_sc as plsc

_LANES = 128
_NEG = -1e30

_S = 2048
_H = 768
_E = 8
_BM = 256
_NB = 23
_NPAD = _NB * _BM
_NCHUNK = 32
_CH = 128
_BS1 = 512
_TOK_W = _S // 32


def _router_body(x_ref, rw_ref, rb_ref, i1_ref, i2_ref, p0_ref, hist_ref):
    logits = jnp.dot(x_ref[...], rw_ref[...],
                     preferred_element_type=jnp.float32) + rb_ref[...]
    lane = jax.lax.broadcasted_iota(jnp.int32, logits.shape, 1)
    m1 = jnp.max(logits, axis=1, keepdims=True)
    i1 = jnp.min(jnp.where(logits == m1, lane, _LANES), axis=1, keepdims=True)
    l2 = jnp.where(lane == i1, _NEG, logits)
    m2 = jnp.max(l2, axis=1, keepdims=True)
    i2 = jnp.min(jnp.where(l2 == m2, lane, _LANES), axis=1, keepdims=True)
    p0 = 1.0 / (1.0 + jnp.exp(m2 - m1))
    i1_ref[...] = i1
    i2_ref[...] = i2
    p0_ref[...] = p0
    m1h = (lane == i1).astype(jnp.int32)
    m2h = (lane == i2).astype(jnp.int32)
    rows = [jnp.sum(m1h[s * _CH:(s + 1) * _CH], axis=0) for s in range(4)]
    rows += [jnp.sum(m2h[s * _CH:(s + 1) * _CH], axis=0) for s in range(4)]
    hist_ref[...] = jnp.stack(rows, axis=0)


def _router(xs, rwp, rbp):
    grid = (_S // _BS1,)
    return pl.pallas_call(
        _router_body,
        grid=grid,
        in_specs=[
            pl.BlockSpec((_BS1, _H), lambda t: (t, 0)),
            pl.BlockSpec((_H, _LANES), lambda t: (0, 0)),
            pl.BlockSpec((1, _LANES), lambda t: (0, 0)),
        ],
        out_specs=[
            pl.BlockSpec((_BS1, 1), lambda t: (t, 0)),
            pl.BlockSpec((_BS1, 1), lambda t: (t, 0)),
            pl.BlockSpec((_BS1, 1), lambda t: (t, 0)),
            pl.BlockSpec((8, _LANES), lambda t: (t, 0)),
        ],
        out_shape=[
            jax.ShapeDtypeStruct((_S, 1), jnp.int32),
            jax.ShapeDtypeStruct((_S, 1), jnp.int32),
            jax.ShapeDtypeStruct((_S, 1), jnp.float32),
            jax.ShapeDtypeStruct((_NCHUNK, _LANES), jnp.int32),
        ],
        compiler_params=pltpu.CompilerParams(
            dimension_semantics=("arbitrary",),
        ),
    )(xs, rwp, rbp)


def _dispatch_body(eids_hbm, hist_hbm, x_hbm,
                   xsort_hbm, pos_hbm, be_hbm,
                   eid_v, hist_v, pos_v, rows_v, be_v, sem):
    wid = lax.axis_index("s") * 2 + lax.axis_index("c")
    slot = (wid % 8) // 4
    t0 = (wid // 8) * _BS1 + (wid % 4) * _CH
    p0off = slot * _S + t0
    lane = lax.iota(jnp.int32, 16)

    pltpu.sync_copy(eids_hbm.at[pl.ds(p0off, _CH)], eid_v)
    pltpu.sync_copy(hist_hbm, hist_v)

    tot = jnp.zeros((16,), jnp.int32)
    run = jnp.zeros((16,), jnp.int32)
    for c in range(_NCHUNK):
        hrow = hist_v[c, pl.ds(0, 16)]
        tot = tot + hrow
        run = run + jnp.where(c < wid, hrow, 0)
    padded = ((tot + (_BM - 1)) >> 8) << 8
    cum = plsc.cumsum(padded)
    start = cum - padded
    base_vec = start + run

    for j in range(_CH // 16):
        ev = eid_v[pl.ds(j * 16, 16)]
        posj = jnp.zeros((16,), jnp.int32)
        for e in range(_E):
            m = ev == e
            csum = plsc.cumsum(m.astype(jnp.int32))
            sp = jnp.sum(jnp.where(lane == e, base_vec, 0))
            posj = jnp.where(m, sp + csum - 1, posj)
            cnts = plsc.all_reduce_population_count(m)
            base_vec = base_vec + jnp.where(lane == e, cnts, 0)
        pos_v[pl.ds(j * 16, 16)] = posj

    pltpu.sync_copy(pos_v, pos_hbm.at[pl.ds(p0off, _CH)])
    pltpu.sync_copy(x_hbm.at[pl.ds(t0, _CH)], rows_v)
    pltpu.async_copy(rows_v, xsort_hbm.at[pos_v], sem).wait()

    @pl.when(wid == 0)
    def _():
        for half in range(2):
            bidx = lane + half * 16
            rowstart = bidx * _BM
            val = jnp.full((16,), -1, jnp.int32)
            for e in range(_E):
                s = jnp.sum(jnp.where(lane == e, start, 0))
                p = jnp.sum(jnp.where(lane == e, padded, 0))
                val = jnp.where((rowstart >= s) & (rowstart < s + p), e, val)
            be_v[pl.ds(half * 16, 16)] = val
        pltpu.sync_copy(be_v, be_hbm)


def _dispatch(eids, hist, xs):
    mesh = plsc.VectorSubcoreMesh(core_axis_name="c", subcore_axis_name="s")
    f = pl.kernel(
        _dispatch_body,
        out_type=[
            jax.ShapeDtypeStruct((_NPAD, _H), jnp.float32),
            jax.ShapeDtypeStruct((2 * _S,), jnp.int32),
            jax.ShapeDtypeStruct((_NCHUNK,), jnp.int32),
        ],
        mesh=mesh,
        scratch_types=[
            pltpu.VMEM((_CH,), jnp.int32),
            pltpu.VMEM((_NCHUNK, _LANES), jnp.int32),
            pltpu.VMEM((_CH,), jnp.int32),
            pltpu.VMEM((_CH, _H), jnp.float32),
            pltpu.VMEM((_NCHUNK,), jnp.int32),
            pltpu.SemaphoreType.DMA,
        ],
        compiler_params=pltpu.CompilerParams(needs_layout_passes=False),
    )
    return f(eids, hist, xs)


def _ffn_body(be_ref, x_ref, w1_ref, b1_ref, w2_ref, b2_ref, out_ref):
    b = pl.program_id(0)
    g = be_ref[b]

    @pl.when(g >= 0)
    def _():
        xb = x_ref[...].astype(jnp.bfloat16)
        h = jax.nn.gelu(
            jnp.dot(xb, w1_ref[g], preferred_element_type=jnp.float32)
            + b1_ref[g])
        out_ref[...] = (jnp.dot(h.astype(jnp.bfloat16), w2_ref[g],
                                preferred_element_type=jnp.float32)
                        + b2_ref[g])


def _ffn(be, xsort, W1b, b1, W2b, b2):
    grid = (_NB,)
    return pl.pallas_call(
        _ffn_body,
        grid=grid,
        in_specs=[
            pl.BlockSpec(memory_space=pltpu.SMEM),
            pl.BlockSpec((_BM, _H), lambda b: (b, 0)),
            pl.BlockSpec((_E, _H, _H), lambda b: (0, 0, 0)),
            pl.BlockSpec((_E, _H), lambda b: (0, 0)),
            pl.BlockSpec((_E, _H, _H), lambda b: (0, 0, 0)),
            pl.BlockSpec((_E, _H), lambda b: (0, 0)),
        ],
        out_specs=pl.BlockSpec((_BM, _H), lambda b: (b, 0)),
        out_shape=jax.ShapeDtypeStruct((_NPAD, _H), jnp.float32),
        compiler_params=pltpu.CompilerParams(
            dimension_semantics=("arbitrary",),
        ),
    )(be, xsort, W1b, b1, W2b, b2)


def _combine_body(osort_hbm, pos_hbm, p0_hbm, out_hbm,
                  idxa_v, idxb_v, p0_v, bufa_v, bufb_v, sema, semb):
    wid = lax.axis_index("s") * 2 + lax.axis_index("c")
    base = wid * _TOK_W
    pltpu.sync_copy(pos_hbm.at[pl.ds(base, _TOK_W)], idxa_v)
    pltpu.sync_copy(pos_hbm.at[pl.ds(_S + base, _TOK_W)], idxb_v)
    pltpu.sync_copy(p0_hbm.at[pl.ds(base, _TOK_W)], p0_v)
    cpa = pltpu.async_copy(osort_hbm.at[idxa_v], bufa_v, sema)
    cpb = pltpu.async_copy(osort_hbm.at[idxb_v], bufb_v, semb)
    cpa.wait()
    cpb.wait()

    def body(r, carry):
        wsp = plsc.load_gather(p0_v, [jnp.full((16,), r, jnp.int32)])
        for c in range(_H // 16):
            a = bufa_v[r, pl.ds(c * 16, 16)]
            bb = bufb_v[r, pl.ds(c * 16, 16)]
            bufa_v[r, pl.ds(c * 16, 16)] = (a + bb) * wsp
        return carry

    lax.fori_loop(0, _TOK_W, body, 0)
    pltpu.sync_copy(bufa_v, out_hbm.at[pl.ds(base, _TOK_W)])


def _combine(osort, pos, p0):
    mesh = plsc.VectorSubcoreMesh(core_axis_name="c", subcore_axis_name="s")
    f = pl.kernel(
        _combine_body,
        out_type=jax.ShapeDtypeStruct((_S, _H), jnp.float32),
        mesh=mesh,
        scratch_types=[
            pltpu.VMEM((_TOK_W,), jnp.int32),
            pltpu.VMEM((_TOK_W,), jnp.int32),
            pltpu.VMEM((_TOK_W,), jnp.float32),
            pltpu.VMEM((_TOK_W, _H), jnp.float32),
            pltpu.VMEM((_TOK_W, _H), jnp.float32),
            pltpu.SemaphoreType.DMA,
            pltpu.SemaphoreType.DMA,
        ],
        compiler_params=pltpu.CompilerParams(needs_layout_passes=False),
    )
    return f(osort, pos, p0)


def kernel(x, training, router_W, router_b, W1, b1, W2, b2):
    B, S, H = x.shape
    E = router_W.shape[1]
    xs = x.reshape(S, H)
    rwp = jnp.pad(router_W, ((0, 0), (0, _LANES - E)))
    rbp = jnp.concatenate(
        [router_b, jnp.full((_LANES - E,), _NEG, router_b.dtype)]
    ).reshape(1, _LANES)
    W1b = W1.astype(jnp.bfloat16)
    W2b = W2.astype(jnp.bfloat16)

    i1, i2, p0, hist = _router(xs, rwp, rbp)
    eids = jnp.concatenate([i1.reshape(S), i2.reshape(S)])
    p0 = p0.reshape(S)
    xsort, pos, be = _dispatch(eids, hist, xs)
    osort = _ffn(be, xsort, W1b, b1, W2b, b2)
    out = _combine(osort, pos, p0)
    return out.reshape(B, S, H)
